# Initial kernel scaffold; baseline (speedup 1.0000x reference)
import jax
import jax.numpy as jnp
from jax.experimental import pallas as pl


def kernel(x, alphas, W, b, fc_W, fc_b, edge_index):
    raise NotImplementedError("write your pallas kernel here")


# speedup vs baseline: 2.3923x; 2.3923x over previous
import functools

import jax
import jax.numpy as jnp
from jax import lax
from jax.experimental import pallas as pl
from jax.experimental.pallas import tpu as pltpu
from jax.experimental.pallas import tpu_sc as plsc

N = 10000
E = 160000
D = 256
DH = 128
NUM_CELLS = 3

NC = 2
NS = 16
CHUNK = 128
E_PAD = 163840
ROWS_PER_TILE = 632
NP = NS * ROWS_PER_TILE


@functools.lru_cache(maxsize=None)
def _sc_kernels():
  _mesh = plsc.VectorSubcoreMesh(
      core_axis_name="c", subcore_axis_name="s", num_cores=NC, num_subcores=NS)

  @functools.partial(
      pl.kernel,
      out_type=jax.ShapeDtypeStruct((NC * NP, DH), jnp.float32),
      mesh=_mesh,
      scratch_types=[
          pltpu.VMEM((CHUNK,), jnp.int32),
          pltpu.VMEM((CHUNK, DH), jnp.float32),
          pltpu.VMEM_SHARED((NP, DH), jnp.float32),
      ],
  )
  def _sc_degree(dst_hbm, ones_hbm, zrow_hbm, out_hbm, didx, ones_v, degsh):
      c = lax.axis_index("c")
      s = lax.axis_index("s")
      wid = s * NC + c
      pltpu.sync_copy(zrow_hbm, degsh.at[pl.ds(s * ROWS_PER_TILE, ROWS_PER_TILE)])
      pltpu.sync_copy(ones_hbm, ones_v)
      plsc.subcore_barrier()
      per_tile = E_PAD // (NC * NS)

      def body(i, carry):
          eb = pl.multiple_of(wid * per_tile + i * CHUNK, CHUNK)
          pltpu.sync_copy(dst_hbm.at[pl.ds(eb, CHUNK)], didx)
          pltpu.sync_copy(ones_v, degsh.at[didx], add=True)
          return carry

      lax.fori_loop(0, per_tile // CHUNK, body, 0)
      plsc.subcore_barrier()
      r0 = s * ROWS_PER_TILE
      pltpu.sync_copy(degsh.at[pl.ds(r0, ROWS_PER_TILE)],
                      out_hbm.at[pl.ds(c * NP + r0, ROWS_PER_TILE)])


  @functools.partial(
        pl.kernel,
        out_type=jax.ShapeDtypeStruct((NC * NP, DH), jnp.float32),
        mesh=_mesh,
        scratch_types=[
            pltpu.VMEM((CHUNK,), jnp.int32),
            pltpu.VMEM((CHUNK,), jnp.int32),
            pltpu.VMEM((CHUNK, DH), jnp.float32),
            pltpu.VMEM_SHARED((NP, DH), jnp.float32),
            pltpu.SemaphoreType.DMA,
        ],
    )
  def _sc_segsum(srcoff_hbm, dst_hbm, h_hbm, zrow_hbm, out_hbm,
                 sidx, didx, rows, aggsh, sem):
      c = lax.axis_index("c")
      s = lax.axis_index("s")
      pltpu.sync_copy(zrow_hbm, aggsh.at[pl.ds(s * ROWS_PER_TILE, ROWS_PER_TILE)])
      plsc.subcore_barrier()
      per_tile = E_PAD // NS

      def body(i, carry):
          eb = pl.multiple_of(s * per_tile + i * CHUNK, CHUNK)
          pltpu.sync_copy(srcoff_hbm.at[pl.ds(c * E_PAD + eb, CHUNK)], sidx)
          pltpu.async_copy(h_hbm.at[sidx], rows, sem).wait()
          pltpu.sync_copy(dst_hbm.at[pl.ds(eb, CHUNK)], didx)
          pltpu.sync_copy(rows, aggsh.at[didx], add=True)
          return carry

      lax.fori_loop(0, per_tile // CHUNK, body, 0)
      plsc.subcore_barrier()
      r0 = s * ROWS_PER_TILE
      pltpu.sync_copy(aggsh.at[pl.ds(r0, ROWS_PER_TILE)],
                      out_hbm.at[pl.ds(c * NP + r0, ROWS_PER_TILE)])

  return _sc_degree, _sc_segsum


_BN = 1000


def _mix(agg_ref, deg_ref, w0_ref, w1_ref, b0_ref, b1_ref, al_ref):
    al = al_ref[...]
    e = jnp.exp(al - jnp.max(al, axis=1, keepdims=True))
    aw = e / jnp.sum(e, axis=1, keepdims=True)
    a0 = aw[:, 0:1]
    a1 = aw[:, 1:2]
    aggf = jnp.concatenate([agg_ref[0], agg_ref[1]], axis=1)
    deg = jnp.maximum(deg_ref[0, :, 0:1] + deg_ref[1, :, 0:1], 1.0)
    op0 = jnp.dot(aggf / deg, w0_ref[...],
                  preferred_element_type=jnp.float32) + b0_ref[...]
    op1 = jnp.dot(aggf, w1_ref[...],
                  preferred_element_type=jnp.float32) + b1_ref[...]
    op1 = jnp.maximum(op1, 0.0)
    return a0 * op0 + a1 * op1


def _dense_mid_body(agg_ref, deg_ref, w0_ref, w1_ref, b0_ref, b1_ref, al_ref,
                    out_ref):
    h = _mix(agg_ref, deg_ref, w0_ref, w1_ref, b0_ref, b1_ref, al_ref)
    out_ref[0] = h[:, :DH]
    out_ref[1] = h[:, DH:]


def _dense_final_body(agg_ref, deg_ref, w0_ref, w1_ref, b0_ref, b1_ref,
                      al_ref, fct_ref, fcb_ref, out_ref, acc_ref):
    i = pl.program_id(0)
    h = _mix(agg_ref, deg_ref, w0_ref, w1_ref, b0_ref, b1_ref, al_ref)
    part = jnp.sum(h, axis=0, keepdims=True)

    @pl.when(i == 0)
    def _():
        acc_ref[...] = part

    @pl.when(i > 0)
    def _():
        acc_ref[...] += part

    @pl.when(i == pl.num_programs(0) - 1)
    def _():
        pooled = acc_ref[...] * (1.0 / N)
        out_ref[...] = (jnp.sum(pooled * fct_ref[...], axis=1, keepdims=True)
                        + fcb_ref[...])


_common_specs = [
    pl.BlockSpec((2, _BN, DH), lambda i: (0, i, 0)),
    pl.BlockSpec((2, _BN, 16), lambda i: (0, i, 0)),
    pl.BlockSpec((D, D), lambda i: (0, 0)),
    pl.BlockSpec((D, D), lambda i: (0, 0)),
    pl.BlockSpec((1, D), lambda i: (0, 0)),
    pl.BlockSpec((1, D), lambda i: (0, 0)),
    pl.BlockSpec((1, 2), lambda i: (0, 0)),
]

_dense_mid = pl.pallas_call(
    _dense_mid_body,
    grid=(N // _BN,),
    in_specs=_common_specs,
    out_specs=pl.BlockSpec((2, _BN, DH), lambda i: (0, i, 0)),
    out_shape=jax.ShapeDtypeStruct((2, N, DH), jnp.float32),
)

_dense_final = pl.pallas_call(
    _dense_final_body,
    grid=(N // _BN,),
    in_specs=_common_specs + [
        pl.BlockSpec((1, D), lambda i: (0, 0)),
        pl.BlockSpec((1, 1), lambda i: (0, 0)),
    ],
    out_specs=pl.BlockSpec((1, 1), lambda i: (0, 0)),
    out_shape=jax.ShapeDtypeStruct((1, 1), jnp.float32),
    scratch_shapes=[pltpu.VMEM((1, D), jnp.float32)],
)


def kernel(x, alphas, W, b, fc_W, fc_b, edge_index):
    src = edge_index[0]
    dst = edge_index[1]
    pad = E_PAD - E
    src_pad = jnp.concatenate([src, jnp.zeros((pad,), jnp.int32)])
    srcoff = jnp.concatenate([src_pad, src_pad + N])
    dst_pad = jnp.concatenate([dst, jnp.full((pad,), N, jnp.int32)])
    zrow = jnp.zeros((ROWS_PER_TILE, DH), jnp.float32)
    ones128 = jnp.ones((CHUNK, DH), jnp.float32)

    sc_degree, sc_segsum = _sc_kernels()
    deg2 = sc_degree(dst_pad, ones128, zrow).reshape(2, NP, DH)[:, :N, :16]

    h = jnp.concatenate([x[:, :DH], x[:, DH:]], axis=0)
    out = None
    for c in range(NUM_CELLS):
        agg = sc_segsum(srcoff, dst_pad, h, zrow).reshape(2, NP, DH)[:, :N, :]
        w0 = W[c, 0]
        w1 = W[c, 1]
        b0 = b[c, 0].reshape(1, D)
        b1 = b[c, 1].reshape(1, D)
        al = alphas[c].reshape(1, 2)
        if c < NUM_CELLS - 1:
            h = _dense_mid(agg, deg2, w0, w1, b0, b1, al).reshape(2 * N, DH)
        else:
            out = _dense_final(agg, deg2, w0, w1, b0, b1, al,
                               fc_W.reshape(1, D), fc_b.reshape(1, 1))
    return out


# speedup vs baseline: 3.1281x; 1.3076x over previous
import functools

import jax
import jax.numpy as jnp
from jax import lax
from jax.experimental import pallas as pl
from jax.experimental.pallas import tpu as pltpu
from jax.experimental.pallas import tpu_sc as plsc

N = 10000
E = 160000
D = 256
DH = 128
NUM_CELLS = 3

NC = 2
NS = 16
CHUNK = 128
E_PAD = 163840
ROWS_PER_TILE = 632
NP = NS * ROWS_PER_TILE
NBUF = 2
IDXH = 40

CPT = E_PAD // NS // CHUNK
CPT_DEG = E_PAD // (NC * NS) // CHUNK


@functools.lru_cache(maxsize=None)
def _sc_kernels():
  _mesh = plsc.VectorSubcoreMesh(
      core_axis_name="c", subcore_axis_name="s", num_cores=NC, num_subcores=NS)

  @functools.partial(
      pl.kernel,
      out_type=jax.ShapeDtypeStruct((NC * NP, DH), jnp.float32),
      mesh=_mesh,
      scratch_types=[
          pltpu.VMEM((CPT_DEG, CHUNK), jnp.int32),
          pltpu.VMEM((CHUNK, DH), jnp.float32),
          pltpu.VMEM_SHARED((NP, DH), jnp.float32),
      ],
  )
  def _sc_degree(dst_hbm, ones_hbm, zrow_hbm, out_hbm, didx, ones_v, degsh):
      c = lax.axis_index("c")
      s = lax.axis_index("s")
      wid = s * NC + c
      pltpu.sync_copy(zrow_hbm, degsh.at[pl.ds(s * ROWS_PER_TILE, ROWS_PER_TILE)])
      pltpu.sync_copy(ones_hbm, ones_v)
      pltpu.sync_copy(dst_hbm.at[pl.ds(wid * CPT_DEG, CPT_DEG)], didx)
      plsc.subcore_barrier()

      def body(j, carry):
          pltpu.sync_copy(ones_v, degsh.at[didx.at[j]], add=True)
          return carry

      lax.fori_loop(0, CPT_DEG, body, 0)
      plsc.subcore_barrier()
      r0 = s * ROWS_PER_TILE
      pltpu.sync_copy(degsh.at[pl.ds(r0, ROWS_PER_TILE)],
                      out_hbm.at[pl.ds(c * NP + r0, ROWS_PER_TILE)])

  @functools.partial(
      pl.kernel,
      out_type=jax.ShapeDtypeStruct((NC * NP, DH), jnp.float32),
      mesh=_mesh,
      scratch_types=[
          pltpu.VMEM((IDXH, CHUNK), jnp.int32),
          pltpu.VMEM((IDXH, CHUNK), jnp.int32),
          pltpu.VMEM((NBUF, CHUNK, DH), jnp.float32),
          pltpu.VMEM_SHARED((NP, DH), jnp.float32),
          [pltpu.SemaphoreType.DMA] * NBUF,
      ],
  )
  def _sc_segsum(srcoff_hbm, dst_hbm, h_hbm, zrow_hbm, out_hbm,
                 sidx, didx, rows, aggsh, sems):
      c = lax.axis_index("c")
      s = lax.axis_index("s")
      pltpu.sync_copy(zrow_hbm, aggsh.at[pl.ds(s * ROWS_PER_TILE, ROWS_PER_TILE)])
      plsc.subcore_barrier()

      for half in range(CPT // IDXH):
          pltpu.sync_copy(
              srcoff_hbm.at[pl.ds((c * NS + s) * CPT + half * IDXH, IDXH)],
              sidx)
          pltpu.sync_copy(dst_hbm.at[pl.ds(s * CPT + half * IDXH, IDXH)], didx)
          for b in range(NBUF):
              pltpu.async_copy(h_hbm.at[sidx.at[b]], rows.at[b], sems[b])

          def body(g, carry):
              for b in range(NBUF):
                  j = g * NBUF + b
                  pltpu.make_async_copy(h_hbm.at[sidx.at[j]],
                                        rows.at[b], sems[b]).wait()
                  pltpu.sync_copy(rows.at[b], aggsh.at[didx.at[j]], add=True)

                  @pl.when(j + NBUF < IDXH)
                  def _():
                      pltpu.async_copy(h_hbm.at[sidx.at[j + NBUF]],
                                       rows.at[b], sems[b])
              return carry

          lax.fori_loop(0, IDXH // NBUF, body, 0)
      plsc.subcore_barrier()
      r0 = s * ROWS_PER_TILE
      pltpu.sync_copy(aggsh.at[pl.ds(r0, ROWS_PER_TILE)],
                      out_hbm.at[pl.ds(c * NP + r0, ROWS_PER_TILE)])

  return _sc_degree, _sc_segsum


_BN = 1000


def _mix(agg_ref, deg_ref, w0_ref, w1_ref, b0_ref, b1_ref, al_ref):
    al = al_ref[...]
    e = jnp.exp(al - jnp.max(al, axis=1, keepdims=True))
    aw = e / jnp.sum(e, axis=1, keepdims=True)
    a0 = aw[:, 0:1]
    a1 = aw[:, 1:2]
    aggf = jnp.concatenate([agg_ref[0], agg_ref[1]], axis=1)
    deg = jnp.maximum(deg_ref[0, :, 0:1] + deg_ref[1, :, 0:1], 1.0)
    op0 = jnp.dot(aggf / deg, w0_ref[...],
                  preferred_element_type=jnp.float32) + b0_ref[...]
    op1 = jnp.dot(aggf, w1_ref[...],
                  preferred_element_type=jnp.float32) + b1_ref[...]
    op1 = jnp.maximum(op1, 0.0)
    return a0 * op0 + a1 * op1


def _dense_mid_body(agg_ref, deg_ref, w0_ref, w1_ref, b0_ref, b1_ref, al_ref,
                    out_ref):
    h = _mix(agg_ref, deg_ref, w0_ref, w1_ref, b0_ref, b1_ref, al_ref)
    out_ref[0] = h[:, :DH]
    out_ref[1] = h[:, DH:]


def _dense_final_body(agg_ref, deg_ref, w0_ref, w1_ref, b0_ref, b1_ref,
                      al_ref, fct_ref, fcb_ref, out_ref, acc_ref):
    i = pl.program_id(0)
    h = _mix(agg_ref, deg_ref, w0_ref, w1_ref, b0_ref, b1_ref, al_ref)
    part = jnp.sum(h, axis=0, keepdims=True)

    @pl.when(i == 0)
    def _():
        acc_ref[...] = part

    @pl.when(i > 0)
    def _():
        acc_ref[...] += part

    @pl.when(i == pl.num_programs(0) - 1)
    def _():
        pooled = acc_ref[...] * (1.0 / N)
        out_ref[...] = (jnp.sum(pooled * fct_ref[...], axis=1, keepdims=True)
                        + fcb_ref[...])


_common_specs = [
    pl.BlockSpec((2, _BN, DH), lambda i: (0, i, 0)),
    pl.BlockSpec((2, _BN, DH), lambda i: (0, i, 0)),
    pl.BlockSpec((D, D), lambda i: (0, 0)),
    pl.BlockSpec((D, D), lambda i: (0, 0)),
    pl.BlockSpec((1, D), lambda i: (0, 0)),
    pl.BlockSpec((1, D), lambda i: (0, 0)),
    pl.BlockSpec((1, 2), lambda i: (0, 0)),
]

_dense_mid = pl.pallas_call(
    _dense_mid_body,
    grid=(N // _BN,),
    in_specs=_common_specs,
    out_specs=pl.BlockSpec((2, _BN, DH), lambda i: (0, i, 0)),
    out_shape=jax.ShapeDtypeStruct((2, N, DH), jnp.float32),
)

_dense_final = pl.pallas_call(
    _dense_final_body,
    grid=(N // _BN,),
    in_specs=_common_specs + [
        pl.BlockSpec((1, D), lambda i: (0, 0)),
        pl.BlockSpec((1, 1), lambda i: (0, 0)),
    ],
    out_specs=pl.BlockSpec((1, 1), lambda i: (0, 0)),
    out_shape=jax.ShapeDtypeStruct((1, 1), jnp.float32),
    scratch_shapes=[pltpu.VMEM((1, D), jnp.float32)],
)


def kernel(x, alphas, W, b, fc_W, fc_b, edge_index):
    src = edge_index[0]
    dst = edge_index[1]
    pad = E_PAD - E
    src_pad = jnp.concatenate([src, jnp.zeros((pad,), jnp.int32)])
    srcoff = jnp.concatenate([src_pad, src_pad + N]).reshape(-1, CHUNK)
    dst_pad = jnp.concatenate(
        [dst, jnp.full((pad,), N, jnp.int32)]).reshape(-1, CHUNK)
    zrow = jnp.zeros((ROWS_PER_TILE, DH), jnp.float32)
    ones128 = jnp.ones((CHUNK, DH), jnp.float32)

    sc_degree, sc_segsum = _sc_kernels()
    deg2 = sc_degree(dst_pad, ones128, zrow).reshape(2, NP, DH)

    h = jnp.concatenate([x[:, :DH], x[:, DH:]], axis=0)
    out = None
    for c in range(NUM_CELLS):
        agg = sc_segsum(srcoff, dst_pad, h, zrow).reshape(2, NP, DH)
        w0 = W[c, 0]
        w1 = W[c, 1]
        b0 = b[c, 0].reshape(1, D)
        b1 = b[c, 1].reshape(1, D)
        al = alphas[c].reshape(1, 2)
        if c < NUM_CELLS - 1:
            h = _dense_mid(agg, deg2, w0, w1, b0, b1, al).reshape(2 * N, DH)
        else:
            out = _dense_final(agg, deg2, w0, w1, b0, b1, al,
                               fc_W.reshape(1, D), fc_b.reshape(1, 1))
    return out


# speedup vs baseline: 3.2081x; 1.0256x over previous
import functools

import jax
import jax.numpy as jnp
from jax import lax
from jax.experimental import pallas as pl
from jax.experimental.pallas import tpu as pltpu
from jax.experimental.pallas import tpu_sc as plsc

N = 10000
E = 160000
D = 256
DH = 128
NUM_CELLS = 3

NC = 2
NS = 16
CHUNK = 128
E_PAD = 163840
ROWS_PER_TILE = 632
NP = NS * ROWS_PER_TILE
NBUF = 2
IDXH = 40

CPT = E_PAD // NS // CHUNK
CPT_DEG = E_PAD // (NC * NS) // CHUNK


@functools.lru_cache(maxsize=None)
def _sc_kernels():
  _mesh = plsc.VectorSubcoreMesh(
      core_axis_name="c", subcore_axis_name="s", num_cores=NC, num_subcores=NS)

  @functools.partial(
      pl.kernel,
      out_type=jax.ShapeDtypeStruct((NC * NP, DH), jnp.float32),
      mesh=_mesh,
      scratch_types=[
          pltpu.VMEM((CPT_DEG, CHUNK), jnp.int32),
          pltpu.VMEM((CHUNK, DH), jnp.float32),
          pltpu.VMEM_SHARED((NP, DH), jnp.float32),
      ],
  )
  def _sc_degree(dst_hbm, ones_hbm, zrow_hbm, out_hbm, didx, ones_v, degsh):
      c = lax.axis_index("c")
      s = lax.axis_index("s")
      wid = s * NC + c
      pltpu.sync_copy(zrow_hbm, degsh.at[pl.ds(s * ROWS_PER_TILE, ROWS_PER_TILE)])
      pltpu.sync_copy(ones_hbm, ones_v)
      pltpu.sync_copy(dst_hbm.at[pl.ds(wid * CPT_DEG, CPT_DEG)], didx)
      plsc.subcore_barrier()

      def body(j, carry):
          pltpu.sync_copy(ones_v, degsh.at[didx.at[j]], add=True)
          return carry

      lax.fori_loop(0, CPT_DEG, body, 0)
      plsc.subcore_barrier()
      r0 = s * ROWS_PER_TILE
      pltpu.sync_copy(degsh.at[pl.ds(r0, ROWS_PER_TILE)],
                      out_hbm.at[pl.ds(c * NP + r0, ROWS_PER_TILE)])

  @functools.partial(
      pl.kernel,
      out_type=jax.ShapeDtypeStruct((NC * NP, DH), jnp.float32),
      mesh=_mesh,
      scratch_types=[
          pltpu.VMEM((IDXH, CHUNK), jnp.int32),
          pltpu.VMEM((IDXH, CHUNK), jnp.int32),
          pltpu.VMEM((NBUF, CHUNK, DH), jnp.float32),
          pltpu.VMEM_SHARED((NP, DH), jnp.float32),
          [pltpu.SemaphoreType.DMA] * NBUF,
      ],
  )
  def _sc_segsum(srcoff_hbm, dst_hbm, h_hbm, zrow_hbm, out_hbm,
                 sidx, didx, rows, aggsh, sems):
      c = lax.axis_index("c")
      s = lax.axis_index("s")
      pltpu.sync_copy(zrow_hbm, aggsh.at[pl.ds(s * ROWS_PER_TILE, ROWS_PER_TILE)])
      plsc.subcore_barrier()
      col = c * DH

      for half in range(CPT // IDXH):
          pltpu.sync_copy(
              srcoff_hbm.at[pl.ds(s * CPT + half * IDXH, IDXH)], sidx)
          pltpu.sync_copy(dst_hbm.at[pl.ds(s * CPT + half * IDXH, IDXH)], didx)
          for b in range(NBUF):
              pltpu.async_copy(h_hbm.at[sidx.at[b], pl.ds(col, DH)],
                               rows.at[b], sems[b])

          def body(g, carry):
              for b in range(NBUF):
                  j = g * NBUF + b
                  pltpu.make_async_copy(h_hbm.at[sidx.at[j], pl.ds(col, DH)],
                                        rows.at[b], sems[b]).wait()
                  pltpu.sync_copy(rows.at[b], aggsh.at[didx.at[j]], add=True)

                  @pl.when(j + NBUF < IDXH)
                  def _():
                      pltpu.async_copy(
                          h_hbm.at[sidx.at[j + NBUF], pl.ds(col, DH)],
                          rows.at[b], sems[b])
              return carry

          lax.fori_loop(0, IDXH // NBUF, body, 0)
      plsc.subcore_barrier()
      r0 = s * ROWS_PER_TILE
      pltpu.sync_copy(aggsh.at[pl.ds(r0, ROWS_PER_TILE)],
                      out_hbm.at[pl.ds(c * NP + r0, ROWS_PER_TILE)])

  return _sc_degree, _sc_segsum


_BN = 1000


def _mix(agg_ref, deg_ref, w0_ref, w1_ref, b0_ref, b1_ref, al_ref):
    al = al_ref[...]
    e = jnp.exp(al - jnp.max(al, axis=1, keepdims=True))
    aw = e / jnp.sum(e, axis=1, keepdims=True)
    a0 = aw[:, 0:1]
    a1 = aw[:, 1:2]
    aggf = jnp.concatenate([agg_ref[0], agg_ref[1]], axis=1)
    deg = jnp.maximum(deg_ref[0, :, 0:1] + deg_ref[1, :, 0:1], 1.0)
    op0 = jnp.dot(aggf / deg, w0_ref[...],
                  preferred_element_type=jnp.float32) + b0_ref[...]
    op1 = jnp.dot(aggf, w1_ref[...],
                  preferred_element_type=jnp.float32) + b1_ref[...]
    op1 = jnp.maximum(op1, 0.0)
    return a0 * op0 + a1 * op1


def _dense_mid_body(agg_ref, deg_ref, w0_ref, w1_ref, b0_ref, b1_ref, al_ref,
                    out_ref):
    out_ref[...] = _mix(agg_ref, deg_ref, w0_ref, w1_ref, b0_ref, b1_ref,
                        al_ref)


def _dense_final_body(agg_ref, deg_ref, w0_ref, w1_ref, b0_ref, b1_ref,
                      al_ref, fct_ref, fcb_ref, out_ref, acc_ref):
    i = pl.program_id(0)
    h = _mix(agg_ref, deg_ref, w0_ref, w1_ref, b0_ref, b1_ref, al_ref)
    part = jnp.sum(h, axis=0, keepdims=True)

    @pl.when(i == 0)
    def _():
        acc_ref[...] = part

    @pl.when(i > 0)
    def _():
        acc_ref[...] += part

    @pl.when(i == pl.num_programs(0) - 1)
    def _():
        pooled = acc_ref[...] * (1.0 / N)
        out_ref[...] = (jnp.sum(pooled * fct_ref[...], axis=1, keepdims=True)
                        + fcb_ref[...])


_common_specs = [
    pl.BlockSpec((2, _BN, DH), lambda i: (0, i, 0)),
    pl.BlockSpec((2, _BN, DH), lambda i: (0, i, 0)),
    pl.BlockSpec((D, D), lambda i: (0, 0)),
    pl.BlockSpec((D, D), lambda i: (0, 0)),
    pl.BlockSpec((1, D), lambda i: (0, 0)),
    pl.BlockSpec((1, D), lambda i: (0, 0)),
    pl.BlockSpec((1, 2), lambda i: (0, 0)),
]

_dense_mid = pl.pallas_call(
    _dense_mid_body,
    grid=(N // _BN,),
    in_specs=_common_specs,
    out_specs=pl.BlockSpec((_BN, D), lambda i: (i, 0)),
    out_shape=jax.ShapeDtypeStruct((N, D), jnp.float32),
)

_dense_final = pl.pallas_call(
    _dense_final_body,
    grid=(N // _BN,),
    in_specs=_common_specs + [
        pl.BlockSpec((1, D), lambda i: (0, 0)),
        pl.BlockSpec((1, 1), lambda i: (0, 0)),
    ],
    out_specs=pl.BlockSpec((1, 1), lambda i: (0, 0)),
    out_shape=jax.ShapeDtypeStruct((1, 1), jnp.float32),
    scratch_shapes=[pltpu.VMEM((1, D), jnp.float32)],
)


def kernel(x, alphas, W, b, fc_W, fc_b, edge_index):
    src = edge_index[0]
    dst = edge_index[1]
    pad = E_PAD - E
    srcoff = jnp.concatenate(
        [src, jnp.zeros((pad,), jnp.int32)]).reshape(-1, CHUNK)
    dst_pad = jnp.concatenate(
        [dst, jnp.full((pad,), N, jnp.int32)]).reshape(-1, CHUNK)
    zrow = jnp.zeros((ROWS_PER_TILE, DH), jnp.float32)
    ones128 = jnp.ones((CHUNK, DH), jnp.float32)

    sc_degree, sc_segsum = _sc_kernels()
    deg2 = sc_degree(dst_pad, ones128, zrow).reshape(2, NP, DH)

    h = x
    out = None
    for c in range(NUM_CELLS):
        agg = sc_segsum(srcoff, dst_pad, h, zrow).reshape(2, NP, DH)
        w0 = W[c, 0]
        w1 = W[c, 1]
        b0 = b[c, 0].reshape(1, D)
        b1 = b[c, 1].reshape(1, D)
        al = alphas[c].reshape(1, 2)
        if c < NUM_CELLS - 1:
            h = _dense_mid(agg, deg2, w0, w1, b0, b1, al)
        else:
            out = _dense_final(agg, deg2, w0, w1, b0, b1, al,
                               fc_W.reshape(1, D), fc_b.reshape(1, 1))
    return out
